# Initial kernel scaffold; baseline (speedup 1.0000x reference)
import jax
import jax.numpy as jnp
from jax.experimental import pallas as pl


def kernel(x, edge_index, weight, params):
    raise NotImplementedError("write your pallas kernel here")


# speedup vs baseline: 4.2443x; 4.2443x over previous
import functools

import jax
import jax.numpy as jnp
from jax import lax
from jax.experimental import pallas as pl
from jax.experimental.pallas import tpu as pltpu
from jax.experimental.pallas import tpu_sc as plsc

_N = 10000
_E = 160000
_NC = 2
_NS = 16
_BLK = 128
_RPT = 80
_EPT = _BLK * _RPT
_EPAD = _EPT * _NS
_ROWS = _EPAD // _BLK
_NP = 10240
_NPT = _NP // _NS


def _cdiv(a, b):
    return (a + b - 1) // b


@functools.lru_cache(maxsize=None)
def _seg_sums(CH, LW):
    mesh = plsc.VectorSubcoreMesh(core_axis_name="c", subcore_axis_name="s")
    rounds = _cdiv(CH, _NC)

    @functools.partial(
        pl.kernel,
        mesh=mesh,
        out_type=jax.ShapeDtypeStruct((CH, _NP, LW), jnp.float32),
        scratch_types=[
            pltpu.VMEM((_EPT,), jnp.int32),
            pltpu.VMEM((_EPT,), jnp.int32),
            pltpu.VMEM((_RPT, _BLK), jnp.int32),
            pltpu.VMEM((_BLK, LW), jnp.float32),
            pltpu.VMEM_SHARED((_NP, LW), jnp.float32),
            pltpu.SemaphoreType.DMA,
        ],
    )
    def k(y2, src1, dst2, zeros, out, srcv, idxv, dstv, rows, acc, sem):
        cid = lax.axis_index("c")
        sid = lax.axis_index("s")
        pltpu.sync_copy(src1.at[pl.ds(sid * _EPT, _EPT)], srcv)
        pltpu.sync_copy(dst2.at[pl.ds(sid * _RPT, _RPT)], dstv)
        for r in range(rounds):
            chunk = cid + _NC * r

            @pl.when(chunk < CH)
            def _():
                pltpu.sync_copy(zeros, acc.at[pl.ds(sid * _NPT, _NPT)])
                def mkidx(g, carry):
                    sv = srcv[pl.ds(g * 16, 16)]
                    idxv[pl.ds(g * 16, 16)] = sv * CH + chunk
                    return carry

                lax.fori_loop(0, _EPT // 16, mkidx, 0)
                plsc.subcore_barrier()

                def blk(j, carry):
                    pltpu.async_copy(
                        y2.at[idxv.at[pl.ds(j * _BLK, _BLK)]], rows, sem
                    ).wait()
                    pltpu.sync_copy(rows, acc.at[dstv.at[j]], add=True)
                    return carry

                lax.fori_loop(0, _RPT, blk, 0)
                plsc.subcore_barrier()
                pltpu.sync_copy(
                    acc.at[pl.ds(sid * _NPT, _NPT)],
                    out.at[chunk, pl.ds(sid * _NPT, _NPT)],
                )
                plsc.subcore_barrier()

    return k


def _tc_layer_body(CH, elu, s3, cnt, h, wl, wr, b, o):
    inv = 1.0 / jnp.maximum(cnt[...], 1.0)
    acc = jnp.dot(h[...], wr[...], preferred_element_type=jnp.float32)
    for c in range(CH):
        acc += jnp.dot(s3[c] * inv, wl[c], preferred_element_type=jnp.float32)
    acc += b[...]
    if elu:
        acc = jnp.where(acc > 0.0, acc, jnp.exp(acc) - 1.0)
    o[...] = acc


def _tc_layer(CH, LW, K, DOUT, elu, s3, cnt, h, wl, wr, b):
    R = 1000
    return pl.pallas_call(
        functools.partial(_tc_layer_body, CH, elu),
        grid=(_N // R,),
        in_specs=[
            pl.BlockSpec((CH, R, LW), lambda i: (0, i, 0)),
            pl.BlockSpec((R, 1), lambda i: (i, 0)),
            pl.BlockSpec((R, K), lambda i: (i, 0)),
            pl.BlockSpec((CH, LW, DOUT), lambda i: (0, 0, 0)),
            pl.BlockSpec((K, DOUT), lambda i: (0, 0)),
            pl.BlockSpec((1, DOUT), lambda i: (0, 0)),
        ],
        out_specs=pl.BlockSpec((R, DOUT), lambda i: (i, 0)),
        out_shape=jax.ShapeDtypeStruct((_N, DOUT), jnp.float32),
    )(s3, cnt, h, wl, wr, b)


def _mm_body(h, w, o):
    o[...] = jnp.dot(h[...], w[...], preferred_element_type=jnp.float32)


def _tc_matmul(K, DOUT, h, w):
    R = 1000
    return pl.pallas_call(
        _mm_body,
        grid=(_N // R,),
        in_specs=[
            pl.BlockSpec((R, K), lambda i: (i, 0)),
            pl.BlockSpec((K, DOUT), lambda i: (0, 0)),
        ],
        out_specs=pl.BlockSpec((R, DOUT), lambda i: (i, 0)),
        out_shape=jax.ShapeDtypeStruct((_N, DOUT), jnp.float32),
    )(h, w)


def kernel(x, edge_index, weight, params):
    del weight
    src = edge_index[0].astype(jnp.int32)
    dst = edge_index[1].astype(jnp.int32)
    pad = _EPAD - _E
    pad_i = jnp.arange(pad, dtype=jnp.int32)
    src1 = jnp.concatenate([src, pad_i % 64])
    dst2 = jnp.concatenate([dst, _N + (pad_i % 8)]).reshape(_ROWS, _BLK)

    zeros128 = jnp.zeros((_NPT, 128), jnp.float32)
    ones128 = jnp.ones((_N, 128), jnp.float32)

    cnt = _seg_sums(1, 128)(ones128, src1, dst2, zeros128)[0, :_N, :1]

    h = x.astype(jnp.float32)
    for i in range(8):
        Wl, Wr, b = params[i]
        K = Wl.shape[0]
        CH = K // 128
        s3 = _seg_sums(CH, 128)(h.reshape(_N * CH, 128), src1, dst2, zeros128)
        h = _tc_layer(CH, 128, K, 512, True, s3, cnt, h,
                      Wl.reshape(CH, 128, 512), Wr, b.reshape(1, 512))

    Wl, Wr, b = params[8]
    wl_pad = jnp.concatenate([Wl, jnp.zeros((512, 64), jnp.float32)], axis=1)
    y = _tc_matmul(512, 128, h, wl_pad)
    s3 = _seg_sums(1, 128)(y, src1, dst2, zeros128)
    eye_pad = jnp.concatenate(
        [jnp.eye(64, dtype=jnp.float32), jnp.zeros((64, 64), jnp.float32)],
        axis=0).reshape(1, 128, 64)
    out = _tc_layer(1, 128, 512, 64, False, s3, cnt, h, eye_pad, Wr,
                    b.reshape(1, 64))
    return out
